# Initial kernel scaffold; baseline (speedup 1.0000x reference)
import jax
import jax.numpy as jnp
from jax.experimental import pallas as pl


def kernel(x, table):
    raise NotImplementedError("write your pallas kernel here")


# speedup vs baseline: 1.5776x; 1.5776x over previous
import functools

import jax
import jax.numpy as jnp
from jax import lax
from jax.experimental import pallas as pl
from jax.experimental.pallas import tpu as pltpu
from jax.experimental.pallas import tpu_sc as plsc

NUM_ROWS = 16384
NUM_COLS = 26
DIM = 32
B_TOTAL = NUM_ROWS * NUM_COLS
NW = 32
B_PER_W = B_TOTAL // NW
CHUNK = 1024
NCHUNK = B_PER_W // CHUNK


def _emb_body(x_hbm, table_hbm, out_hbm, idx_v, rows_v, gsem, ssem):
    wid = lax.axis_index("s") * 2 + lax.axis_index("c")
    base = wid * B_PER_W
    pltpu.sync_copy(x_hbm.at[pl.ds(base, B_PER_W)], idx_v)

    def gather(g, buf):
        pltpu.async_copy(
            table_hbm.at[idx_v.at[pl.ds(g * CHUNK, CHUNK)]], rows_v.at[buf], gsem
        )

    def store(g, buf):
        pltpu.async_copy(
            rows_v.at[buf], out_hbm.at[pl.ds(base + g * CHUNK, CHUNK)], ssem
        )

    gather(0, 0)

    def body(g, _):
        buf = lax.rem(g, 2)
        nbuf = lax.rem(g + 1, 2)

        @pl.when(g + 1 < NCHUNK)
        def _():
            gather(g + 1, nbuf)

        pltpu.make_async_copy(
            table_hbm.at[idx_v.at[pl.ds(g * CHUNK, CHUNK)]], rows_v.at[buf], gsem
        ).wait()

        @pl.when(g >= 2)
        def _():
            pltpu.make_async_copy(
                rows_v.at[buf], out_hbm.at[pl.ds(base + (g - 2) * CHUNK, CHUNK)], ssem
            ).wait()

        store(g, buf)
        return 0

    lax.fori_loop(0, NCHUNK, body, 0)
    pltpu.make_async_copy(
        rows_v.at[(NCHUNK - 2) % 2],
        out_hbm.at[pl.ds(base + (NCHUNK - 2) * CHUNK, CHUNK)],
        ssem,
    ).wait()
    pltpu.make_async_copy(
        rows_v.at[(NCHUNK - 1) % 2],
        out_hbm.at[pl.ds(base + (NCHUNK - 1) * CHUNK, CHUNK)],
        ssem,
    ).wait()


@jax.jit
def _embedding_lookup(x_flat, table):
    mesh = plsc.VectorSubcoreMesh(core_axis_name="c", subcore_axis_name="s")
    run = functools.partial(
        pl.kernel,
        mesh=mesh,
        compiler_params=pltpu.CompilerParams(use_tc_tiling_on_sc=False),
        out_type=jax.ShapeDtypeStruct((B_TOTAL, DIM), jnp.float32),
        scratch_types=[
            pltpu.VMEM((B_PER_W,), jnp.int32),
            pltpu.VMEM((2, CHUNK, DIM), jnp.float32),
            pltpu.SemaphoreType.DMA,
            pltpu.SemaphoreType.DMA,
        ],
    )(_emb_body)
    return run(x_flat, table)


def kernel(x, table):
    out = _embedding_lookup(x.reshape(-1), table)
    return out.reshape(NUM_ROWS, NUM_COLS, DIM)


# speedup vs baseline: 1.5778x; 1.0001x over previous
import functools

import jax
import jax.numpy as jnp
from jax import lax
from jax.experimental import pallas as pl
from jax.experimental.pallas import tpu as pltpu
from jax.experimental.pallas import tpu_sc as plsc

NUM_ROWS = 16384
NUM_COLS = 26
DIM = 32
NUM_EMB = 1000000
B_TOTAL = NUM_ROWS * NUM_COLS
NW = 32
B_PER_W = B_TOTAL // NW
CHUNK = 1024
NCHUNK = B_PER_W // CHUNK


def _emb_body(x_hbm, table_hbm, out_hbm, idx_v, rows_v, gsem, ssem):
    wid = lax.axis_index("s") * 2 + lax.axis_index("c")
    base = wid * B_PER_W
    pltpu.sync_copy(x_hbm.at[pl.ds(base, B_PER_W)], idx_v)

    def gather(g, buf):
        pltpu.async_copy(
            table_hbm.at[idx_v.at[pl.ds(g * CHUNK, CHUNK)]], rows_v.at[buf], gsem
        )

    def store(g, buf):
        pltpu.async_copy(
            rows_v.at[buf], out_hbm.at[pl.ds(base + g * CHUNK, CHUNK)], ssem
        )

    gather(0, 0)

    def body(g, _):
        buf = lax.rem(g, 2)
        nbuf = lax.rem(g + 1, 2)

        @pl.when(g + 1 < NCHUNK)
        def _():
            gather(g + 1, nbuf)

        pltpu.make_async_copy(
            table_hbm.at[idx_v.at[pl.ds(g * CHUNK, CHUNK)]], rows_v.at[buf], gsem
        ).wait()

        @pl.when(g >= 2)
        def _():
            pltpu.make_async_copy(
                rows_v.at[buf], out_hbm.at[pl.ds(base + (g - 2) * CHUNK, CHUNK)], ssem
            ).wait()

        store(g, buf)
        return 0

    lax.fori_loop(0, NCHUNK, body, 0)
    pltpu.make_async_copy(
        rows_v.at[(NCHUNK - 2) % 2],
        out_hbm.at[pl.ds(base + (NCHUNK - 2) * CHUNK, CHUNK)],
        ssem,
    ).wait()
    pltpu.make_async_copy(
        rows_v.at[(NCHUNK - 1) % 2],
        out_hbm.at[pl.ds(base + (NCHUNK - 1) * CHUNK, CHUNK)],
        ssem,
    ).wait()


@jax.jit
def _embedding_lookup(x_flat, table):
    mesh = plsc.VectorSubcoreMesh(core_axis_name="c", subcore_axis_name="s")
    run = functools.partial(
        pl.kernel,
        mesh=mesh,
        compiler_params=pltpu.CompilerParams(use_tc_tiling_on_sc=False),
        out_type=jax.ShapeDtypeStruct((B_TOTAL, DIM), jnp.float32),
        scratch_types=[
            pltpu.VMEM((B_PER_W,), jnp.int32),
            pltpu.VMEM((2, CHUNK, DIM), jnp.float32),
            pltpu.SemaphoreType.DMA,
            pltpu.SemaphoreType.DMA,
        ],
    )(_emb_body)
    return run(x_flat, table)


def kernel(x, table):
    t128 = jax.lax.optimization_barrier(table.reshape(NUM_EMB // 4, DIM * 4))
    out = _embedding_lookup(x.reshape(-1), t128.reshape(NUM_EMB, DIM))
    return out.reshape(NUM_ROWS, NUM_COLS, DIM)
